# baseline (speedup 1.0000x reference)
import functools

import jax
import jax.numpy as jnp
from jax import lax
from jax.experimental import pallas as pl
from jax.experimental.pallas import tpu as pltpu
from jax.experimental.pallas import tpu_sc as plsc

_B = 16384
_D = 64
_LAMDA = 0.5
_LAMDA_T = 0.25
_NC = 2
_NS = 16
_NW = _NC * _NS
_BPW = _B // _NW
_G = _BPW // 16


def _rsqrt(x):
    i = lax.bitcast_convert_type(x, jnp.int32)
    i = jnp.int32(0x5F3759DF) - (i >> 1)
    y = lax.bitcast_convert_type(i, jnp.float32)
    for _ in range(3):
        y = y * (1.5 - 0.5 * x * y * y)
    return y


def _body(lp_hbm, iu_hbm, tu_hbm, du_hbm, di_hbm, bu_hbm, bi_hbm,
          pqu_hbm, pqi_hbm, ywu_hbm, ywi_hbm, rdeg_hbm, tdeg_hbm,
          out_hbm,
          idxu_v, idxi_v, iu_v, tu_v, lp_v, bu_v, bi_v, rdeg_v, tdeg_v,
          wpu_v, wit_v, wwv_v, buf0, buf1, stage_v,
          sem_a, sem0, sem1):
    wid = lax.axis_index("s") * _NC + lax.axis_index("c")
    base = wid * _BPW

    pltpu.sync_copy(du_hbm.at[pl.ds(base, _BPW)], idxu_v)
    pltpu.sync_copy(di_hbm.at[pl.ds(base, _BPW)], idxi_v)
    pltpu.sync_copy(iu_hbm.at[pl.ds(base, _BPW)], iu_v)
    pltpu.sync_copy(tu_hbm.at[pl.ds(base, _BPW)], tu_v)
    pltpu.sync_copy(lp_hbm.at[pl.ds(base, _BPW)], lp_v)

    c_bu = pltpu.async_copy(bu_hbm.at[idxu_v], bu_v, sem_a)
    c_bi = pltpu.async_copy(bi_hbm.at[idxi_v], bi_v, sem_a)
    c_rd = pltpu.async_copy(rdeg_hbm.at[idxi_v], rdeg_v, sem_a)
    c_td = pltpu.async_copy(tdeg_hbm.at[idxu_v], tdeg_v, sem_a)

    big0 = pltpu.async_copy(pqu_hbm.at[idxu_v], buf0, sem0)
    big1 = pltpu.async_copy(pqi_hbm.at[idxi_v], buf1, sem1)

    c_bu.wait()
    c_bi.wait()
    c_rd.wait()
    c_td.wait()

    def wgroup(g, carry):
        reg, lnk = carry
        sl = pl.ds(g * 16, 16)
        iu = iu_v[sl]
        tu = tu_v[sl]
        bu = bu_v[sl]
        bi = bi_v[sl]
        lpv = lp_v[sl]
        rdeg = rdeg_v[sl]
        tdeg = tdeg_v[sl]
        uj = jnp.where(rdeg > 0,
                       _rsqrt(jnp.maximum(rdeg.astype(jnp.float32), 1.0)), 0.0)
        tv = jnp.where(tdeg > 0,
                       _rsqrt(jnp.maximum(tdeg.astype(jnp.float32), 1.0)), 0.0)
        w_it = _LAMDA * uj
        wit_v[sl] = w_it
        wpu_v[sl] = _LAMDA * iu + _LAMDA_T * tu
        wwv_v[sl] = _LAMDA_T * tv
        reg = reg + (_LAMDA * iu) * (bu * bu) + w_it * (bi * bi)
        d = lpv - 1.0
        lnk = lnk + d * d
        return (reg, lnk)

    zero = jnp.zeros((16,), jnp.float32)
    reg, lnk = lax.fori_loop(0, _G, wgroup, (zero, zero))

    def table_pass(buf, w_ref, acc):
        def gbody(g, a):
            a0, a1, a2, a3 = a
            wv = w_ref[pl.ds(g * 16, 16)]
            for j in range(16):
                r = g * 16 + j
                w = wv[j]
                v0 = buf[r, pl.ds(0, 16)]
                v1 = buf[r, pl.ds(16, 16)]
                v2 = buf[r, pl.ds(32, 16)]
                v3 = buf[r, pl.ds(48, 16)]
                a0 = a0 + w * (v0 * v0)
                a1 = a1 + w * (v1 * v1)
                a2 = a2 + w * (v2 * v2)
                a3 = a3 + w * (v3 * v3)
            return (a0, a1, a2, a3)
        return lax.fori_loop(0, _G, gbody, acc)

    acc4 = (zero, zero, zero, zero)
    big0.wait()
    acc4 = table_pass(buf0, wpu_v, acc4)
    big2 = pltpu.async_copy(ywi_hbm.at[idxi_v], buf0, sem0)
    big1.wait()
    acc4 = table_pass(buf1, wit_v, acc4)
    big3 = pltpu.async_copy(ywu_hbm.at[idxu_v], buf1, sem1)
    big2.wait()
    acc4 = table_pass(buf0, wit_v, acc4)
    big3.wait()
    acc4 = table_pass(buf1, wwv_v, acc4)

    reg = reg + acc4[0] + acc4[1] + acc4[2] + acc4[3]

    stage_v[...] = reg
    pltpu.sync_copy(stage_v, out_hbm.at[2 * wid])
    stage_v[...] = lnk
    pltpu.sync_copy(stage_v, out_hbm.at[2 * wid + 1])


_run = functools.partial(
    pl.kernel,
    mesh=plsc.VectorSubcoreMesh(core_axis_name="c", subcore_axis_name="s"),
    out_type=jax.ShapeDtypeStruct((2 * _NW, 16), jnp.float32),
    compiler_params=pltpu.CompilerParams(use_tc_tiling_on_sc=False),
    scratch_types=[
        pltpu.VMEM((_BPW,), jnp.int32),
        pltpu.VMEM((_BPW,), jnp.int32),
        pltpu.VMEM((_BPW,), jnp.float32),
        pltpu.VMEM((_BPW,), jnp.float32),
        pltpu.VMEM((_BPW,), jnp.float32),
        pltpu.VMEM((_BPW,), jnp.float32),
        pltpu.VMEM((_BPW,), jnp.float32),
        pltpu.VMEM((_BPW,), jnp.int32),
        pltpu.VMEM((_BPW,), jnp.int32),
        pltpu.VMEM((_BPW,), jnp.float32),
        pltpu.VMEM((_BPW,), jnp.float32),
        pltpu.VMEM((_BPW,), jnp.float32),
        pltpu.VMEM((_BPW, _D), jnp.float32),
        pltpu.VMEM((_BPW, _D), jnp.float32),
        pltpu.VMEM((16,), jnp.float32),
        pltpu.SemaphoreType.DMA,
        pltpu.SemaphoreType.DMA,
        pltpu.SemaphoreType.DMA,
    ],
)(_body)


def kernel(link_pred, bias_user, bias_item, p_q_user, p_q_item, y_w_user,
           y_w_item, I_u_factor, T_u_factor, dst_user, dst_item,
           rated_by_deg, trusted_by_deg):
    out = _run(
        link_pred,
        I_u_factor.reshape(-1),
        T_u_factor.reshape(-1),
        dst_user.astype(jnp.int32),
        dst_item.astype(jnp.int32),
        bias_user.reshape(-1),
        bias_item.reshape(-1),
        p_q_user, p_q_item, y_w_user, y_w_item,
        rated_by_deg.astype(jnp.int32),
        trusted_by_deg.astype(jnp.int32),
    )
    reg_loss = jnp.sum(out[0::2]) / _B
    link_loss = _LAMDA_T * (jnp.sum(out[1::2]) / _B)
    return (reg_loss, link_loss)


# speedup vs baseline: 1.2553x; 1.2553x over previous
import functools

import jax
import jax.numpy as jnp
from jax import lax
from jax.experimental import pallas as pl
from jax.experimental.pallas import tpu as pltpu
from jax.experimental.pallas import tpu_sc as plsc

_B = 16384
_D = 64
_LAMDA = 0.5
_LAMDA_T = 0.25
_NC = 2
_NS = 16
_NW = _NC * _NS
_BPW = _B // _NW
_G = _BPW // 16
_HC = _BPW // 2


def _rsqrt(x):
    i = lax.bitcast_convert_type(x, jnp.int32)
    i = jnp.int32(0x5F3759DF) - (i >> 1)
    y = lax.bitcast_convert_type(i, jnp.float32)
    for _ in range(3):
        y = y * (1.5 - 0.5 * x * y * y)
    return y


def _body(lp_hbm, iu_hbm, tu_hbm, du_hbm, di_hbm, bu_hbm, bi_hbm,
          pqu_hbm, pqi_hbm, ywu_hbm, ywi_hbm, rdeg_hbm, tdeg_hbm,
          out_hbm,
          idxu_v, idxi_v, iu_v, tu_v, lp_v, bu_v, bi_v, rdeg_v, tdeg_v,
          wpu_v, wit_v, wwv_v, buf0, buf1, stage_v,
          sem_a, sem_b, sem0, sem1):
    wid = lax.axis_index("s") * _NC + lax.axis_index("c")
    base = wid * _BPW

    pltpu.sync_copy(du_hbm.at[pl.ds(base, _BPW)], idxu_v)
    pltpu.sync_copy(di_hbm.at[pl.ds(base, _BPW)], idxi_v)
    pltpu.sync_copy(iu_hbm.at[pl.ds(base, _BPW)], iu_v)
    pltpu.sync_copy(tu_hbm.at[pl.ds(base, _BPW)], tu_v)
    pltpu.sync_copy(lp_hbm.at[pl.ds(base, _BPW)], lp_v)

    g_rd = pltpu.async_copy(rdeg_hbm.at[idxi_v], rdeg_v, sem_a)
    g_td = pltpu.async_copy(tdeg_hbm.at[idxu_v], tdeg_v, sem_a)
    g_bu = pltpu.async_copy(bu_hbm.at[idxu_v], bu_v, sem_b)
    g_bi = pltpu.async_copy(bi_hbm.at[idxi_v], bi_v, sem_b)

    def issue_rows(tbl, idx_ref, dst, sem, row0):
        def gbody(g, c):
            iv = idx_ref[pl.ds(row0 + g * 16, 16)]
            for j in range(16):
                r = g * 16 + j
                pltpu.async_copy(tbl.at[pl.ds(iv[j], 1)],
                                 dst.at[pl.ds(r, 1)], sem)
            return c
        lax.fori_loop(0, _HC // 16, gbody, 0)

    def drain_buf(tbl, buf, sem):
        pltpu.make_async_copy(tbl.at[pl.ds(0, _HC)], buf, sem).wait()

    issue_rows(pqu_hbm, idxu_v, buf0, sem0, 0)
    issue_rows(pqu_hbm, idxu_v, buf1, sem1, _HC)

    g_rd.wait()
    g_td.wait()
    g_bu.wait()
    g_bi.wait()

    lane = lax.iota(jnp.int32, 16)

    def wgroup(g, carry):
        reg, lnk = carry
        sl = pl.ds(g * 16, 16)
        iu = iu_v[sl]
        tu = tu_v[sl]
        bu = bu_v[sl]
        bi = bi_v[sl]
        lpv = lp_v[sl]
        rdeg = rdeg_v[sl]
        tdeg = tdeg_v[sl]
        uj = jnp.where(rdeg > 0,
                       _rsqrt(jnp.maximum(rdeg.astype(jnp.float32), 1.0)), 0.0)
        tv = jnp.where(tdeg > 0,
                       _rsqrt(jnp.maximum(tdeg.astype(jnp.float32), 1.0)), 0.0)
        w_it = _LAMDA * uj
        wit_v[sl] = w_it
        wpu_v[sl] = _LAMDA * iu + _LAMDA_T * tu
        wwv_v[sl] = _LAMDA_T * tv
        reg = reg + (_LAMDA * iu) * (bu * bu) + w_it * (bi * bi)
        d = lpv - 1.0
        lnk = lnk + d * d
        return (reg, lnk)

    zero = jnp.zeros((16,), jnp.float32)
    reg, lnk = lax.fori_loop(0, _G, wgroup, (zero, zero))

    def table_pass(buf, w_ref, w_off, acc):
        def gbody(g, a):
            a0, a1, a2, a3 = a
            wv = w_ref[pl.ds(w_off + g * 16, 16)]
            for j in range(16):
                r = g * 16 + j
                w = wv[j]
                rows = jnp.full((16,), r, jnp.int32)
                v0 = plsc.load_gather(buf, [rows, lane])
                v1 = plsc.load_gather(buf, [rows, lane + 16])
                v2 = plsc.load_gather(buf, [rows, lane + 32])
                v3 = plsc.load_gather(buf, [rows, lane + 48])
                a0 = a0 + w * (v0 * v0)
                a1 = a1 + w * (v1 * v1)
                a2 = a2 + w * (v2 * v2)
                a3 = a3 + w * (v3 * v3)
            return (a0, a1, a2, a3)
        return lax.fori_loop(0, _HC // 16, gbody, acc)

    tasks = [
        (pqu_hbm, idxu_v, wpu_v, 0), (pqu_hbm, idxu_v, wpu_v, _HC),
        (pqi_hbm, idxi_v, wit_v, 0), (pqi_hbm, idxi_v, wit_v, _HC),
        (ywi_hbm, idxi_v, wit_v, 0), (ywi_hbm, idxi_v, wit_v, _HC),
        (ywu_hbm, idxu_v, wwv_v, 0), (ywu_hbm, idxu_v, wwv_v, _HC),
    ]
    bufs = (buf0, buf1)
    sems = (sem0, sem1)
    acc4 = (zero, zero, zero, zero)
    for t, (tbl, idx_ref, w_ref, row0) in enumerate(tasks):
        buf, sem = bufs[t % 2], sems[t % 2]
        drain_buf(tbl, buf, sem)
        acc4 = table_pass(buf, w_ref, row0, acc4)
        if t + 2 < len(tasks):
            ntbl, nidx, _, nrow0 = tasks[t + 2]
            issue_rows(ntbl, nidx, buf, sem, nrow0)

    reg = reg + acc4[0] + acc4[1] + acc4[2] + acc4[3]

    stage_v[...] = reg
    pltpu.sync_copy(stage_v, out_hbm.at[pl.ds(wid * 32, 16)])
    stage_v[...] = lnk
    pltpu.sync_copy(stage_v, out_hbm.at[pl.ds(wid * 32 + 16, 16)])


_run = functools.partial(
    pl.kernel,
    mesh=plsc.VectorSubcoreMesh(core_axis_name="c", subcore_axis_name="s"),
    out_type=jax.ShapeDtypeStruct((_NW * 32,), jnp.float32),
    compiler_params=pltpu.CompilerParams(
        use_tc_tiling_on_sc=True, needs_layout_passes=False),
    scratch_types=[
        pltpu.VMEM((_BPW,), jnp.int32),
        pltpu.VMEM((_BPW,), jnp.int32),
        pltpu.VMEM((_BPW,), jnp.float32),
        pltpu.VMEM((_BPW,), jnp.float32),
        pltpu.VMEM((_BPW,), jnp.float32),
        pltpu.VMEM((_BPW,), jnp.float32),
        pltpu.VMEM((_BPW,), jnp.float32),
        pltpu.VMEM((_BPW,), jnp.int32),
        pltpu.VMEM((_BPW,), jnp.int32),
        pltpu.VMEM((_BPW,), jnp.float32),
        pltpu.VMEM((_BPW,), jnp.float32),
        pltpu.VMEM((_BPW,), jnp.float32),
        pltpu.VMEM((_HC, _D), jnp.float32),
        pltpu.VMEM((_HC, _D), jnp.float32),
        pltpu.VMEM((16,), jnp.float32),
        pltpu.SemaphoreType.DMA,
        pltpu.SemaphoreType.DMA,
        pltpu.SemaphoreType.DMA,
        pltpu.SemaphoreType.DMA,
    ],
)(_body)


def kernel(link_pred, bias_user, bias_item, p_q_user, p_q_item, y_w_user,
           y_w_item, I_u_factor, T_u_factor, dst_user, dst_item,
           rated_by_deg, trusted_by_deg):
    out = _run(
        link_pred,
        I_u_factor.reshape(-1),
        T_u_factor.reshape(-1),
        dst_user.astype(jnp.int32),
        dst_item.astype(jnp.int32),
        bias_user.reshape(-1),
        bias_item.reshape(-1),
        p_q_user, p_q_item, y_w_user, y_w_item,
        rated_by_deg.astype(jnp.int32),
        trusted_by_deg.astype(jnp.int32),
    )
    o = out.reshape(_NW, 2, 16)
    reg_loss = jnp.sum(o[:, 0, :]) / _B
    link_loss = _LAMDA_T * (jnp.sum(o[:, 1, :]) / _B)
    return (reg_loss, link_loss)
